# baseline (speedup 1.0000x reference)
import functools

import jax
import jax.numpy as jnp
from jax import lax
from jax.experimental import pallas as pl
from jax.experimental.pallas import tpu as pltpu
from jax.experimental.pallas import tpu_sc as plsc

B, C, D = 1024, 1000, 64
DP = 128
NW = 32
RPW = B // NW
BIG = 3.0e38
BLK = 256


def _argmax_body(pred_ref, j0_ref):
    pred = pred_ref[...]
    y0 = jnp.max(pred, axis=1, keepdims=True)
    cols = lax.broadcasted_iota(jnp.int32, (BLK, C), 1)
    cand = jnp.where(pred == y0, cols, 2**30)
    j0_ref[...] = jnp.min(cand, axis=1, keepdims=True)


def _tc_argmax(pred):
    return pl.pallas_call(
        _argmax_body,
        grid=(B // BLK,),
        in_specs=[pl.BlockSpec((BLK, C), lambda i: (i, 0))],
        out_specs=pl.BlockSpec((BLK, 1), lambda i: (i, 0)),
        out_shape=jax.ShapeDtypeStruct((B, 1), jnp.int32),
    )(pred)


def _sc_body(j0_hbm, w_hbm, wj_hbm, idx_v, rows_v, sem):
    wid = lax.axis_index("s") * 2 + lax.axis_index("c")
    base = wid * RPW
    pltpu.sync_copy(j0_hbm.at[pl.ds(base, RPW)], idx_v)
    pltpu.async_copy(w_hbm.at[idx_v], rows_v, sem).wait()
    pltpu.sync_copy(rows_v, wj_hbm.at[pl.ds(base, RPW)])


@functools.lru_cache(maxsize=1)
def _sc_gather():
    return pl.kernel(
        _sc_body,
        out_type=jax.ShapeDtypeStruct((B, DP), jnp.float32),
        mesh=plsc.VectorSubcoreMesh(core_axis_name="c", subcore_axis_name="s"),
        compiler_params=pltpu.CompilerParams(needs_layout_passes=False),
        scratch_types=[
            pltpu.VMEM((RPW,), jnp.int32),
            pltpu.VMEM((RPW, DP), jnp.float32),
            pltpu.SemaphoreType.DMA,
        ],
    )


def _ratio_body(pred_ref, wt_ref, wj_ref, k_ref, out_ref):
    pred = pred_ref[...]
    y0 = jnp.max(pred, axis=1, keepdims=True)
    cols = lax.broadcasted_iota(jnp.int32, (BLK, C), 1)
    cand = jnp.where(pred == y0, cols, 2**30)
    j0 = jnp.min(cand, axis=1, keepdims=True)
    margins = y0 - pred
    wt = wt_ref[...]
    wj = wj_ref[...]
    g = jnp.dot(wj, wt, preferred_element_type=jnp.float32)
    nj = jnp.sum(wj * wj, axis=1, keepdims=True)
    nc = jnp.sum(wt * wt, axis=0, keepdims=True)
    d2 = jnp.maximum(nj + nc - 2.0 * g, 0.0)
    dist = jnp.sqrt(d2) * k_ref[0, 0]
    is_j0 = cols == j0
    ratio = jnp.where(is_j0, BIG, margins / jnp.where(is_j0, 1.0, dist))
    out_ref[...] = jnp.min(ratio, axis=1, keepdims=True)


def _tc_ratios(pred, wt, wj, k_smem):
    return pl.pallas_call(
        _ratio_body,
        grid=(B // BLK,),
        in_specs=[
            pl.BlockSpec((BLK, C), lambda i: (i, 0)),
            pl.BlockSpec((DP, C), lambda i: (0, 0)),
            pl.BlockSpec((BLK, DP), lambda i: (i, 0)),
            pl.BlockSpec(memory_space=pltpu.SMEM),
        ],
        out_specs=pl.BlockSpec((BLK, 1), lambda i: (i, 0)),
        out_shape=jax.ShapeDtypeStruct((B, 1), jnp.float32),
    )(pred, wt, wj, k_smem)


@jax.jit
def kernel(prediction, target, W, K):
    del target
    j0 = _tc_argmax(prediction)
    w_pad = jnp.pad(W, ((0, 0), (0, DP - D)))
    wj = _sc_gather()(j0.reshape(B), w_pad)
    wt = jnp.pad(W.T, ((0, DP - D), (0, 0)))
    out = _tc_ratios(prediction, wt, wj, K.reshape(1, 1))
    return out[:, 0]


# speedup vs baseline: 1.1329x; 1.1329x over previous
import functools

import jax
import jax.numpy as jnp
from jax import lax
from jax.experimental import pallas as pl
from jax.experimental.pallas import tpu as pltpu
from jax.experimental.pallas import tpu_sc as plsc

B, C, D = 1024, 1000, 64
DP = 128
NW = 32
RPW = B // NW
BIG = 3.0e38
BLK = 256


def _argmax_body(pred_ref, j0_ref):
    pred = pred_ref[...]
    y0 = jnp.max(pred, axis=1, keepdims=True)
    cols = lax.broadcasted_iota(jnp.int32, (BLK, C), 1)
    cand = jnp.where(pred == y0, cols, 2**30)
    j0 = jnp.min(cand, axis=1, keepdims=True)
    j0_ref[...] = j0.T


def _tc_argmax(pred):
    return pl.pallas_call(
        _argmax_body,
        grid=(B // BLK,),
        in_specs=[pl.BlockSpec((BLK, C), lambda i: (i, 0))],
        out_specs=pl.BlockSpec((1, BLK), lambda i: (0, i)),
        out_shape=jax.ShapeDtypeStruct((1, B), jnp.int32),
    )(pred)


def _sc_body(j0_hbm, w_hbm, wj_hbm, idx_v, rows_v, sem):
    wid = lax.axis_index("s") * 2 + lax.axis_index("c")
    base = wid * RPW
    pltpu.sync_copy(j0_hbm.at[0, pl.ds(base, RPW)], idx_v)
    pltpu.async_copy(w_hbm.at[idx_v], rows_v, sem).wait()
    pltpu.sync_copy(rows_v, wj_hbm.at[pl.ds(base, RPW)])


@functools.lru_cache(maxsize=1)
def _sc_gather():
    return pl.kernel(
        _sc_body,
        out_type=jax.ShapeDtypeStruct((B, DP), jnp.float32),
        mesh=plsc.VectorSubcoreMesh(core_axis_name="c", subcore_axis_name="s"),
        compiler_params=pltpu.CompilerParams(needs_layout_passes=False),
        scratch_types=[
            pltpu.VMEM((RPW,), jnp.int32),
            pltpu.VMEM((RPW, DP), jnp.float32),
            pltpu.SemaphoreType.DMA,
        ],
    )


def _ratio_body(pred_ref, wt_ref, wj_ref, k_ref, out_ref):
    pred = pred_ref[...]
    y0 = jnp.max(pred, axis=1, keepdims=True)
    cols = lax.broadcasted_iota(jnp.int32, (BLK, C), 1)
    cand = jnp.where(pred == y0, cols, 2**30)
    j0 = jnp.min(cand, axis=1, keepdims=True)
    margins = y0 - pred
    wt = wt_ref[...]
    wj = wj_ref[...]
    g = jnp.dot(wj, wt, preferred_element_type=jnp.float32)
    nj = jnp.sum(wj * wj, axis=1, keepdims=True)
    nc = jnp.sum(wt * wt, axis=0, keepdims=True)
    d2 = jnp.maximum(nj + nc - 2.0 * g, 0.0)
    dist = jnp.sqrt(d2) * k_ref[0, 0]
    is_j0 = cols == j0
    ratio = jnp.where(is_j0, BIG, margins / jnp.where(is_j0, 1.0, dist))
    out_ref[...] = jnp.min(ratio, axis=1, keepdims=True).T


def _tc_ratios(pred, wt, wj, k_smem):
    return pl.pallas_call(
        _ratio_body,
        grid=(B // BLK,),
        in_specs=[
            pl.BlockSpec((BLK, C), lambda i: (i, 0)),
            pl.BlockSpec((DP, C), lambda i: (0, 0)),
            pl.BlockSpec((BLK, DP), lambda i: (i, 0)),
            pl.BlockSpec(memory_space=pltpu.SMEM),
        ],
        out_specs=pl.BlockSpec((1, BLK), lambda i: (0, i)),
        out_shape=jax.ShapeDtypeStruct((1, B), jnp.float32),
    )(pred, wt, wj, k_smem)


@jax.jit
def kernel(prediction, target, W, K):
    del target
    j0 = _tc_argmax(prediction)
    w_pad = jnp.pad(W, ((0, 0), (0, DP - D)))
    wj = _sc_gather()(j0, w_pad)
    wt = jnp.pad(W.T, ((0, DP - D), (0, 0)))
    out = _tc_ratios(prediction, wt, wj, K.reshape(1, 1))
    return out[0]


# speedup vs baseline: 1.1360x; 1.0027x over previous
import functools

import jax
import jax.numpy as jnp
from jax import lax
from jax.experimental import pallas as pl
from jax.experimental.pallas import tpu as pltpu
from jax.experimental.pallas import tpu_sc as plsc

B, C, D = 1024, 1000, 64
DP = 128
NW = 32
RPW = B // NW
BIG = 3.0e38
BLK = 256
CP = 1024
NEG = -1.0e30


def _argmax_body(pred_ref, j0_ref):
    pred = pred_ref[...]
    y0 = jnp.max(pred, axis=1, keepdims=True)
    cols = lax.broadcasted_iota(jnp.int32, (BLK, CP), 1)
    cand = jnp.where(pred == y0, cols, 2**30)
    j0 = jnp.min(cand, axis=1, keepdims=True)
    j0_ref[...] = j0.T


def _tc_argmax(pred):
    return pl.pallas_call(
        _argmax_body,
        grid=(B // BLK,),
        in_specs=[pl.BlockSpec((BLK, CP), lambda i: (i, 0))],
        out_specs=pl.BlockSpec((1, BLK), lambda i: (0, i)),
        out_shape=jax.ShapeDtypeStruct((1, B), jnp.int32),
    )(pred)


def _sc_body(j0_hbm, w_hbm, wj_hbm, idx_v, rows_v, sem):
    wid = lax.axis_index("s") * 2 + lax.axis_index("c")
    base = wid * RPW
    pltpu.sync_copy(j0_hbm.at[0, pl.ds(base, RPW)], idx_v)
    pltpu.async_copy(w_hbm.at[idx_v], rows_v, sem).wait()
    pltpu.sync_copy(rows_v, wj_hbm.at[pl.ds(base, RPW)])


@functools.lru_cache(maxsize=1)
def _sc_gather():
    return pl.kernel(
        _sc_body,
        out_type=jax.ShapeDtypeStruct((B, DP), jnp.float32),
        mesh=plsc.VectorSubcoreMesh(core_axis_name="c", subcore_axis_name="s"),
        compiler_params=pltpu.CompilerParams(needs_layout_passes=False),
        scratch_types=[
            pltpu.VMEM((RPW,), jnp.int32),
            pltpu.VMEM((RPW, DP), jnp.float32),
            pltpu.SemaphoreType.DMA,
        ],
    )


def _ratio_body(pred_ref, wt_ref, wj_ref, k_ref, out_ref):
    pred = pred_ref[...]
    y0 = jnp.max(pred, axis=1, keepdims=True)
    cols = lax.broadcasted_iota(jnp.int32, (BLK, CP), 1)
    cand = jnp.where(pred == y0, cols, 2**30)
    j0 = jnp.min(cand, axis=1, keepdims=True)
    margins = y0 - pred
    wt = wt_ref[...]
    wj = wj_ref[...]
    g = jnp.dot(wj, wt, preferred_element_type=jnp.float32)
    nj = jnp.sum(wj * wj, axis=1, keepdims=True)
    nc = jnp.sum(wt * wt, axis=0, keepdims=True)
    d2 = jnp.maximum(nj + nc - 2.0 * g, 0.0)
    is_j0 = cols == j0
    q = jnp.where(is_j0, BIG,
                  (margins * margins) / jnp.where(is_j0, 1.0, d2))
    qmin = jnp.min(q, axis=1, keepdims=True)
    out_ref[...] = (jnp.sqrt(qmin) / k_ref[0, 0]).T


def _tc_ratios(pred, wt, wj, k_smem):
    return pl.pallas_call(
        _ratio_body,
        grid=(B // BLK,),
        in_specs=[
            pl.BlockSpec((BLK, CP), lambda i: (i, 0)),
            pl.BlockSpec((DP, CP), lambda i: (0, 0)),
            pl.BlockSpec((BLK, DP), lambda i: (i, 0)),
            pl.BlockSpec(memory_space=pltpu.SMEM),
        ],
        out_specs=pl.BlockSpec((1, BLK), lambda i: (0, i)),
        out_shape=jax.ShapeDtypeStruct((1, B), jnp.float32),
    )(pred, wt, wj, k_smem)


@jax.jit
def kernel(prediction, target, W, K):
    del target
    pred_pad = jnp.pad(prediction, ((0, 0), (0, CP - C)),
                       constant_values=NEG)
    j0 = _tc_argmax(pred_pad)
    w_pad = jnp.pad(W, ((0, 0), (0, DP - D)))
    wj = _sc_gather()(j0, w_pad)
    wt = jnp.pad(W.T, ((0, DP - D), (0, CP - C)))
    out = _tc_ratios(pred_pad, wt, wj, K.reshape(1, 1))
    return out[0]


# speedup vs baseline: 1.3249x; 1.1663x over previous
import functools

import jax
import jax.numpy as jnp
from jax import lax
from jax.experimental import pallas as pl
from jax.experimental.pallas import tpu as pltpu
from jax.experimental.pallas import tpu_sc as plsc

B, C, D = 1024, 1000, 64
DP = 128
NW = 32
RPW = B // NW
BIG = 3.0e38
BLK = 256


def _argmax_body(predt_ref, j0_ref):
    predt = predt_ref[...]
    y0 = jnp.max(predt, axis=0, keepdims=True)
    rows = lax.broadcasted_iota(jnp.int32, (C, BLK), 0)
    cand = jnp.where(predt == y0, rows, 2**30)
    j0_ref[...] = jnp.min(cand, axis=0, keepdims=True)


def _tc_argmax(predt):
    return pl.pallas_call(
        _argmax_body,
        grid=(B // BLK,),
        in_specs=[pl.BlockSpec((C, BLK), lambda i: (0, i))],
        out_specs=pl.BlockSpec((1, BLK), lambda i: (0, i)),
        out_shape=jax.ShapeDtypeStruct((1, B), jnp.int32),
    )(predt)


def _sc_body(j0_hbm, w_hbm, wj_hbm, idx_v, rows_v, sem):
    wid = lax.axis_index("s") * 2 + lax.axis_index("c")
    base = wid * RPW
    pltpu.sync_copy(j0_hbm.at[0, pl.ds(base, RPW)], idx_v)
    pltpu.async_copy(w_hbm.at[idx_v], rows_v, sem).wait()
    pltpu.sync_copy(rows_v, wj_hbm.at[pl.ds(base, RPW)])


@functools.lru_cache(maxsize=1)
def _sc_gather():
    return pl.kernel(
        _sc_body,
        out_type=jax.ShapeDtypeStruct((B, DP), jnp.float32),
        mesh=plsc.VectorSubcoreMesh(core_axis_name="c", subcore_axis_name="s"),
        compiler_params=pltpu.CompilerParams(needs_layout_passes=False),
        scratch_types=[
            pltpu.VMEM((RPW,), jnp.int32),
            pltpu.VMEM((RPW, DP), jnp.float32),
            pltpu.SemaphoreType.DMA,
        ],
    )


def _ratio_body(predt_ref, w_ref, wj_ref, k_ref, out_ref):
    predt = predt_ref[...]
    y0 = jnp.max(predt, axis=0, keepdims=True)
    rows = lax.broadcasted_iota(jnp.int32, (C, BLK), 0)
    cand = jnp.where(predt == y0, rows, 2**30)
    j0 = jnp.min(cand, axis=0, keepdims=True)
    margins = y0 - predt
    w = w_ref[...]
    wj = wj_ref[...][:, :D]
    gt = lax.dot_general(w, wj, (((1,), (1,)), ((), ())),
                         preferred_element_type=jnp.float32)
    nc = jnp.sum(w * w, axis=1, keepdims=True)
    nj = lax.dot_general(jnp.ones((1, D), jnp.float32), wj * wj,
                         (((1,), (1,)), ((), ())),
                         preferred_element_type=jnp.float32)
    d2 = jnp.maximum(nc + nj - 2.0 * gt, 0.0)
    is_j0 = rows == j0
    q = jnp.where(is_j0, BIG,
                  (margins * margins) / jnp.where(is_j0, 1.0, d2))
    qmin = jnp.min(q, axis=0, keepdims=True)
    out_ref[...] = jnp.sqrt(qmin) / k_ref[0, 0]


def _tc_ratios(predt, w, wj, k_smem):
    return pl.pallas_call(
        _ratio_body,
        grid=(B // BLK,),
        in_specs=[
            pl.BlockSpec((C, BLK), lambda i: (0, i)),
            pl.BlockSpec((C, D), lambda i: (0, 0)),
            pl.BlockSpec((BLK, DP), lambda i: (i, 0)),
            pl.BlockSpec(memory_space=pltpu.SMEM),
        ],
        out_specs=pl.BlockSpec((1, BLK), lambda i: (0, i)),
        out_shape=jax.ShapeDtypeStruct((1, B), jnp.float32),
    )(predt, w, wj, k_smem)


@jax.jit
def kernel(prediction, target, W, K):
    del target
    predt = prediction.T
    j0 = _tc_argmax(predt)
    w_pad = jnp.pad(W, ((0, 0), (0, DP - D)))
    wj = _sc_gather()(j0, w_pad)
    out = _tc_ratios(predt, W, wj, K.reshape(1, 1))
    return out[0]


# speedup vs baseline: 1.3533x; 1.0215x over previous
import functools

import jax
import jax.numpy as jnp
from jax import lax
from jax.experimental import pallas as pl
from jax.experimental.pallas import tpu as pltpu
from jax.experimental.pallas import tpu_sc as plsc

B, C, D = 1024, 1000, 64
DP = 128
NW = 32
RPW = B // NW
BIG = 3.0e38
BLK = 256


def _argmax_body(predt_ref, j0_ref):
    predt = predt_ref[...]
    y0 = jnp.max(predt, axis=0, keepdims=True)
    rows = lax.broadcasted_iota(jnp.int32, (C, BLK), 0)
    cand = jnp.where(predt == y0, rows, 2**30)
    j0_ref[...] = jnp.min(cand, axis=0, keepdims=True)


def _tc_argmax(predt):
    return pl.pallas_call(
        _argmax_body,
        grid=(B // BLK,),
        in_specs=[pl.BlockSpec((C, BLK), lambda i: (0, i))],
        out_specs=pl.BlockSpec((1, BLK), lambda i: (0, i)),
        out_shape=jax.ShapeDtypeStruct((1, B), jnp.int32),
    )(predt)


def _sc_body(j0_hbm, w_hbm, wj_hbm, idx_v, rows_v, sem):
    wid = lax.axis_index("s") * 2 + lax.axis_index("c")
    base = wid * RPW
    pltpu.sync_copy(j0_hbm.at[0, pl.ds(base, RPW)], idx_v)
    pltpu.async_copy(w_hbm.at[idx_v], rows_v, sem).wait()
    pltpu.sync_copy(rows_v, wj_hbm.at[pl.ds(base, RPW)])


@functools.lru_cache(maxsize=1)
def _sc_gather():
    return pl.kernel(
        _sc_body,
        out_type=jax.ShapeDtypeStruct((B, DP), jnp.float32),
        mesh=plsc.VectorSubcoreMesh(core_axis_name="c", subcore_axis_name="s"),
        compiler_params=pltpu.CompilerParams(needs_layout_passes=False),
        scratch_types=[
            pltpu.VMEM((RPW,), jnp.int32),
            pltpu.VMEM((RPW, DP), jnp.float32),
            pltpu.SemaphoreType.DMA,
        ],
    )


def _ratio_body(predt_ref, wt_ref, wj_ref, k_ref, out_ref):
    predt = predt_ref[...]
    y0 = jnp.max(predt, axis=0, keepdims=True)
    rows = lax.broadcasted_iota(jnp.int32, (C, BLK), 0)
    cand = jnp.where(predt == y0, rows, 2**30)
    j0 = jnp.min(cand, axis=0, keepdims=True)
    margins = y0 - predt
    wt = wt_ref[...]
    wj = wj_ref[...][:, :D]
    gt = lax.dot_general(wt, wj, (((0,), (1,)), ((), ())),
                         preferred_element_type=jnp.float32)
    ones = jnp.ones((1, D), jnp.float32)
    nc = lax.dot_general(wt * wt, ones, (((0,), (1,)), ((), ())),
                         preferred_element_type=jnp.float32)
    nj = lax.dot_general(ones, wj * wj, (((1,), (1,)), ((), ())),
                         preferred_element_type=jnp.float32)
    d2 = jnp.maximum(nc + nj - 2.0 * gt, 0.0)
    is_j0 = rows == j0
    q = jnp.where(is_j0, BIG,
                  (margins * margins) / jnp.where(is_j0, 1.0, d2))
    qmin = jnp.min(q, axis=0, keepdims=True)
    out_ref[...] = jnp.sqrt(qmin) / k_ref[0, 0]


def _tc_ratios(predt, wt, wj, k_smem):
    return pl.pallas_call(
        _ratio_body,
        grid=(B // BLK,),
        in_specs=[
            pl.BlockSpec((C, BLK), lambda i: (0, i)),
            pl.BlockSpec((D, C), lambda i: (0, 0)),
            pl.BlockSpec((BLK, DP), lambda i: (i, 0)),
            pl.BlockSpec(memory_space=pltpu.SMEM),
        ],
        out_specs=pl.BlockSpec((1, BLK), lambda i: (0, i)),
        out_shape=jax.ShapeDtypeStruct((1, B), jnp.float32),
    )(predt, wt, wj, k_smem)


@jax.jit
def kernel(prediction, target, W, K):
    del target
    predt = prediction.T
    j0 = _tc_argmax(predt)
    w_pad = jnp.pad(W, ((0, 0), (0, DP - D)))
    wj = _sc_gather()(j0, w_pad)
    out = _tc_ratios(predt, W.T, wj, K.reshape(1, 1))
    return out[0]
